# Initial kernel scaffold; baseline (speedup 1.0000x reference)
import functools

import jax
import jax.numpy as jnp
from jax import lax
from jax.experimental import pallas as pl
from jax.experimental.pallas import tpu as pltpu
from jax.experimental.pallas import tpu_sc as plsc

N = 10000
E = 320000
D = 128
DE = 16
B = 16
A = 32

NSC = 2
NTILE = 16
EPC = E // NSC
EPT = EPC // NTILE
C = 80
NCH = EPT // C
RPT = N // NTILE
RZ = 125

NB = 1000
NBLK = N // NB
EB = 4000
EBLK = E // EB

_f32 = jnp.float32


def _prep_body(x_ref, wam_ref, wvm_ref, g_ref, wag_ref, wvg_ref,
               ya_ref, yv_ref, ha_ref, hv_ref):
    x = x_ref[...]
    ya_ref[...] = jnp.dot(x, wam_ref[...], preferred_element_type=_f32)
    yv_ref[...] = jnp.dot(x, wvm_ref[...], preferred_element_type=_f32)

    @pl.when(pl.program_id(0) == 0)
    def _():
        g = g_ref[...]
        ha_ref[...] = jnp.dot(g, wag_ref[...], preferred_element_type=_f32)
        hv_ref[...] = jnp.dot(g, wvg_ref[...], preferred_element_type=_f32)


def _prep(x, wam, wvm, g, wag, wvg):
    full = lambda s: pl.BlockSpec(s, lambda i: tuple(0 for _ in s))
    return pl.pallas_call(
        _prep_body,
        grid=(NBLK,),
        in_specs=[
            pl.BlockSpec((NB, D), lambda i: (i, 0)),
            full((D, D)), full((D, D)), full((B, D)), full((D, D)), full((D, D)),
        ],
        out_specs=[
            pl.BlockSpec((NB, D), lambda i: (i, 0)),
            pl.BlockSpec((NB, D), lambda i: (i, 0)),
            full((B, D)), full((B, D)),
        ],
        out_shape=[
            jax.ShapeDtypeStruct((N, D), _f32),
            jax.ShapeDtypeStruct((N, D), _f32),
            jax.ShapeDtypeStruct((B, D), _f32),
            jax.ShapeDtypeStruct((B, D), _f32),
        ],
    )(x, wam, wvm, g, wag, wvg)


def _eproj_body(ea_ref, wae_ref, wve_ref, ta_ref, tv_ref):
    ea = ea_ref[...]
    ta_ref[...] = jnp.dot(ea, wae_ref[...], preferred_element_type=_f32)
    tv_ref[...] = jnp.dot(ea, wve_ref[...], preferred_element_type=_f32)


def _eproj(ea, wae, wve):
    full = lambda s: pl.BlockSpec(s, lambda i: tuple(0 for _ in s))
    return pl.pallas_call(
        _eproj_body,
        grid=(EBLK,),
        in_specs=[
            pl.BlockSpec((EB, DE), lambda i: (i, 0)),
            full((DE, D)), full((DE, D)),
        ],
        out_specs=[
            pl.BlockSpec((EB, D), lambda i: (i, 0)),
            pl.BlockSpec((EB, D), lambda i: (i, 0)),
        ],
        out_shape=[
            jax.ShapeDtypeStruct((E, D), _f32),
            jax.ShapeDtypeStruct((E, D), _f32),
        ],
    )(ea, wae, wve)


def _sc_body(ei, ya, yv, ta, tv, pa, pv, dg,
             idx_s, idx_d, rows, trow, ones, zbuf, zbuf_d, acc_sh, deg_sh):
    cid = lax.axis_index("c")
    sid = lax.axis_index("s")
    zero16 = jnp.zeros((16,), _f32)
    e0 = (lax.iota(jnp.int32, (16,)) == 0).astype(_f32)

    @pl.loop(0, RZ)
    def _(i):
        for j in range(D // 16):
            zbuf[i, pl.ds(j * 16, 16)] = zero16

    @pl.loop(0, RPT)
    def _(i):
        zbuf_d[i, :] = zero16

    @pl.loop(0, C)
    def _(i):
        ones[i, :] = e0

    row0 = sid * RPT
    for k in range(RPT // RZ):
        pltpu.sync_copy(zbuf, acc_sh.at[pl.ds(row0 + k * RZ, RZ)])
    pltpu.sync_copy(zbuf_d, deg_sh.at[pl.ds(row0, RPT)])
    plsc.subcore_barrier()

    ebase = cid * EPC + sid * EPT

    def edge_pass(y_hbm, t_hbm, with_deg):
        @pl.loop(0, NCH)
        def _(c):
            base = ebase + c * C
            pltpu.sync_copy(ei.at[0, pl.ds(base, C)], idx_s)
            pltpu.sync_copy(ei.at[1, pl.ds(base, C)], idx_d)
            pltpu.sync_copy(y_hbm.at[idx_s], rows)
            pltpu.sync_copy(t_hbm.at[pl.ds(base, C)], trow)

            @pl.loop(0, C)
            def _(i):
                for j in range(D // 16):
                    s = pl.ds(j * 16, 16)
                    rows[i, s] = jnp.maximum(rows[i, s] + trow[i, s], 0.0)

            pltpu.sync_copy(rows, acc_sh.at[idx_d], add=True)
            if with_deg:
                pltpu.sync_copy(ones, deg_sh.at[idx_d], add=True)

    edge_pass(ya, ta, True)
    plsc.subcore_barrier()
    for k in range(RPT // RZ):
        sl = pl.ds(row0 + k * RZ, RZ)
        pltpu.sync_copy(acc_sh.at[sl], pa.at[cid, sl])
    pltpu.sync_copy(deg_sh.at[pl.ds(row0, RPT)], dg.at[cid, pl.ds(row0, RPT)])
    for k in range(RPT // RZ):
        pltpu.sync_copy(zbuf, acc_sh.at[pl.ds(row0 + k * RZ, RZ)])
    plsc.subcore_barrier()
    edge_pass(yv, tv, False)
    plsc.subcore_barrier()
    for k in range(RPT // RZ):
        sl = pl.ds(row0 + k * RZ, RZ)
        pltpu.sync_copy(acc_sh.at[sl], pv.at[cid, sl])


_sc_edges = pl.kernel(
    _sc_body,
    out_type=(
        jax.ShapeDtypeStruct((NSC, N, D), _f32),
        jax.ShapeDtypeStruct((NSC, N, D), _f32),
        jax.ShapeDtypeStruct((NSC, N, 16), _f32),
    ),
    mesh=plsc.VectorSubcoreMesh(core_axis_name="c", subcore_axis_name="s"),
    scratch_types=[
        pltpu.VMEM((C,), jnp.int32),
        pltpu.VMEM((C,), jnp.int32),
        pltpu.VMEM((C, D), _f32),
        pltpu.VMEM((C, D), _f32),
        pltpu.VMEM((C, 16), _f32),
        pltpu.VMEM((RZ, D), _f32),
        pltpu.VMEM((RPT, 16), _f32),
        pltpu.VMEM_SHARED((N, D), _f32),
        pltpu.VMEM_SHARED((N, 16), _f32),
    ],
)


def _post_body(x_ref, pa_ref, pv_ref, dg_ref, b_ref,
               was_ref, wvs_ref, ha_ref, hv_ref, g_ref,
               war_ref, wvr_ref, wp_ref, wo_ref, wval_ref,
               al_ref, nl_ref, val_ref, acc_a, acc_v, cnt_s):
    i = pl.program_id(0)

    @pl.when(i == 0)
    def _():
        acc_a[...] = jnp.zeros_like(acc_a)
        acc_v[...] = jnp.zeros_like(acc_v)
        cnt_s[...] = jnp.zeros_like(cnt_s)

    x = x_ref[...]
    d = dg_ref[...]
    recip = 1.0 / jnp.maximum(d[0, :, 0:1] + d[1, :, 0:1], 1.0)
    bidx = b_ref[...]
    oh = (bidx == lax.broadcasted_iota(jnp.int32, (1, B), 1)).astype(_f32)
    onesm = jnp.ones((NB, D), _f32)
    dn = (((0,), (0,)), ((), ()))

    agg_a = (pa_ref[0] + pa_ref[1]) * recip
    x2a = jnp.maximum(
        jnp.dot(x, was_ref[...], preferred_element_type=_f32) + agg_a
        + jnp.dot(oh, ha_ref[...], preferred_element_type=_f32), 0.0)
    acc_a[...] += lax.dot_general(oh, x2a, dn, preferred_element_type=_f32)
    nl_ref[...] = jnp.dot(x2a, wo_ref[...], preferred_element_type=_f32)

    agg_v = (pv_ref[0] + pv_ref[1]) * recip
    x2v = jnp.maximum(
        jnp.dot(x, wvs_ref[...], preferred_element_type=_f32) + agg_v
        + jnp.dot(oh, hv_ref[...], preferred_element_type=_f32), 0.0)
    acc_v[...] += lax.dot_general(oh, x2v, dn, preferred_element_type=_f32)

    cnt_s[...] += lax.dot_general(oh, onesm, dn, preferred_element_type=_f32)

    @pl.when(i == NBLK - 1)
    def _():
        cnt = jnp.maximum(cnt_s[...], 1.0)
        g = g_ref[...]
        ga = jnp.maximum(
            jnp.dot(acc_a[...] / cnt, war_ref[...], preferred_element_type=_f32) + g, 0.0)
        al_ref[...] = jnp.dot(ga, wp_ref[...], preferred_element_type=_f32)
        gv = jnp.maximum(
            jnp.dot(acc_v[...] / cnt, wvr_ref[...], preferred_element_type=_f32) + g, 0.0)
        val_ref[...] = jnp.dot(gv, wval_ref[...], preferred_element_type=_f32)


def _post(x, pa, pv, dg, bidx2, was, wvs, ha, hv, g, war, wvr, wp, wo, wval):
    full = lambda s: pl.BlockSpec(s, lambda i: tuple(0 for _ in s))
    return pl.pallas_call(
        _post_body,
        grid=(NBLK,),
        in_specs=[
            pl.BlockSpec((NB, D), lambda i: (i, 0)),
            pl.BlockSpec((NSC, NB, D), lambda i: (0, i, 0)),
            pl.BlockSpec((NSC, NB, D), lambda i: (0, i, 0)),
            pl.BlockSpec((NSC, NB, 16), lambda i: (0, i, 0)),
            pl.BlockSpec((NB, 1), lambda i: (i, 0)),
            full((D, D)), full((D, D)), full((B, D)), full((B, D)), full((B, D)),
            full((D, D)), full((D, D)), full((D, A)), full((D, 1)), full((D, 1)),
        ],
        out_specs=[
            full((B, A)),
            pl.BlockSpec((NB, 1), lambda i: (i, 0)),
            full((B, 1)),
        ],
        out_shape=[
            jax.ShapeDtypeStruct((B, A), _f32),
            jax.ShapeDtypeStruct((N, 1), _f32),
            jax.ShapeDtypeStruct((B, 1), _f32),
        ],
        scratch_shapes=[
            pltpu.VMEM((B, D), _f32),
            pltpu.VMEM((B, D), _f32),
            pltpu.VMEM((B, D), _f32),
        ],
    )(x, pa, pv, dg, bidx2, was, wvs, ha, hv, g, war, wvr, wp, wo, wval)


def kernel(latent_nodes, latent_global, edge_attr, edge_index, batch_idx,
           Wa_msg, Wa_edge, Wa_self, Wa_glob, Wa_g, W_pred, W_obj,
           Wv_msg, Wv_edge, Wv_self, Wv_glob, Wv_g, W_val):
    x = latent_nodes.astype(_f32)
    g = latent_global.astype(_f32)
    ea = edge_attr.astype(_f32)
    ei = edge_index.astype(jnp.int32)
    bidx2 = batch_idx.astype(jnp.int32).reshape(N, 1)

    ya, yv, ha, hv = _prep(x, Wa_msg, Wv_msg, g, Wa_glob, Wv_glob)
    ta, tv = _eproj(ea, Wa_edge, Wv_edge)
    pa, pv, dg = _sc_edges(ei, ya, yv, ta, tv)
    al, nl, val = _post(x, pa, pv, dg, bidx2, Wa_self, Wv_self, ha, hv, g,
                        Wa_g, Wv_g, W_pred, W_obj, W_val)
    return (al, nl[:, 0], val[:, 0])


# speedup vs baseline: 2.1714x; 2.1714x over previous
import dataclasses
import functools

import jax
import jax.numpy as jnp
from jax import lax
from jax.experimental import pallas as pl
from jax.experimental.pallas import tpu as pltpu
from jax.experimental.pallas import tpu_sc as plsc

N = 10000
E = 320000
D = 128
DE = 16
B = 16
A = 32

NSC = 2
NTILE = 16
EPC = E // NSC
EPT = EPC // NTILE
C = 80
NCH = EPT // C
RS = 80
NSTR = N // RS
KMAX = (NSTR + NTILE - 1) // NTILE

NB = 1000
NBLK = N // NB
EB = 4000
EBLK = E // EB

_f32 = jnp.float32


def _prep_body(x_ref, wam_ref, wvm_ref, g_ref, wag_ref, wvg_ref,
               ya_ref, yv_ref, ha_ref, hv_ref):
    x = x_ref[...]
    ya_ref[...] = jnp.dot(x, wam_ref[...], preferred_element_type=_f32)
    yv_ref[...] = jnp.dot(x, wvm_ref[...], preferred_element_type=_f32)

    @pl.when(pl.program_id(0) == 0)
    def _():
        g = g_ref[...]
        ha_ref[...] = jnp.dot(g, wag_ref[...], preferred_element_type=_f32)
        hv_ref[...] = jnp.dot(g, wvg_ref[...], preferred_element_type=_f32)


def _prep(x, wam, wvm, g, wag, wvg):
    full = lambda s: pl.BlockSpec(s, lambda i: tuple(0 for _ in s))
    return pl.pallas_call(
        _prep_body,
        grid=(NBLK,),
        in_specs=[
            pl.BlockSpec((NB, D), lambda i: (i, 0)),
            full((D, D)), full((D, D)), full((B, D)), full((D, D)), full((D, D)),
        ],
        out_specs=[
            pl.BlockSpec((NB, D), lambda i: (i, 0)),
            pl.BlockSpec((NB, D), lambda i: (i, 0)),
            full((B, D)), full((B, D)),
        ],
        out_shape=[
            jax.ShapeDtypeStruct((N, D), _f32),
            jax.ShapeDtypeStruct((N, D), _f32),
            jax.ShapeDtypeStruct((B, D), _f32),
            jax.ShapeDtypeStruct((B, D), _f32),
        ],
    )(x, wam, wvm, g, wag, wvg)


def _eproj_body(ea_ref, wae_ref, wve_ref, ta_ref, tv_ref):
    ea = ea_ref[...]
    ta_ref[...] = jnp.dot(ea, wae_ref[...], preferred_element_type=_f32)
    tv_ref[...] = jnp.dot(ea, wve_ref[...], preferred_element_type=_f32)


def _eproj(ea, wae, wve):
    full = lambda s: pl.BlockSpec(s, lambda i: tuple(0 for _ in s))
    return pl.pallas_call(
        _eproj_body,
        grid=(EBLK,),
        in_specs=[
            pl.BlockSpec((EB, DE), lambda i: (i, 0)),
            full((DE, D)), full((DE, D)),
        ],
        out_specs=[
            pl.BlockSpec((EB, D), lambda i: (i, 0)),
            pl.BlockSpec((EB, D), lambda i: (i, 0)),
        ],
        out_shape=[
            jax.ShapeDtypeStruct((E, D), _f32),
            jax.ShapeDtypeStruct((E, D), _f32),
        ],
    )(ea, wae, wve)


def _sc_body(src, dst, ya, yv, ta, tv, pa, pv, dg,
             idx_s, idx_d, rows, trow, acc_sh):
    cid = lax.axis_index("c")
    sid = lax.axis_index("s")
    zero16 = jnp.zeros((16,), _f32)
    e0 = (lax.iota(jnp.int32, 16) == 0).astype(_f32)

    def zero_rows():
        @pl.loop(0, RS)
        def _(i):
            for j in range(D // 16):
                rows[i, pl.ds(j * 16, 16)] = zero16

    def for_stripes(fn):
        for k in range(KMAX):
            stripe = sid + NTILE * k

            @pl.when(stripe < NSTR)
            def _():
                fn(stripe * RS)

    def zero_acc(r0):
        pltpu.sync_copy(rows, acc_sh.at[pl.ds(r0, RS)])

    zero_rows()
    for_stripes(zero_acc)
    plsc.subcore_barrier()

    ebase = cid * EPC + sid * EPT

    def edge_pass(y_hbm, t_hbm):
        @pl.loop(0, NCH)
        def _(c):
            base = ebase + c * C
            pltpu.sync_copy(src.at[pl.ds(base, C)], idx_s)
            pltpu.sync_copy(dst.at[pl.ds(base, C)], idx_d)
            pltpu.sync_copy(y_hbm.at[idx_s], rows)
            pltpu.sync_copy(t_hbm.at[pl.ds(base, C)], trow)

            @pl.loop(0, C)
            def _(i):
                for j in range(D // 16):
                    sl = pl.ds(j * 16, 16)
                    rows[i, sl] = jnp.maximum(rows[i, sl] + trow[i, sl], 0.0)

            pltpu.sync_copy(rows, acc_sh.at[idx_d], add=True)

    def readout(out_hbm):
        def cp(r0):
            sl = pl.ds(r0, RS)
            pltpu.sync_copy(acc_sh.at[sl], out_hbm.at[cid, sl])

        for_stripes(cp)

    edge_pass(ya, ta)
    plsc.subcore_barrier()
    readout(pa)
    zero_rows()
    for_stripes(zero_acc)
    plsc.subcore_barrier()

    edge_pass(yv, tv)
    plsc.subcore_barrier()
    readout(pv)
    zero_rows()
    for_stripes(zero_acc)
    plsc.subcore_barrier()

    @pl.loop(0, RS)
    def _(i):
        trow[i, pl.ds(0, 16)] = e0
        for j in range(1, D // 16):
            trow[i, pl.ds(j * 16, 16)] = zero16

    @pl.loop(0, NCH)
    def _(c):
        base = ebase + c * C
        pltpu.sync_copy(dst.at[pl.ds(base, C)], idx_d)
        pltpu.sync_copy(trow, acc_sh.at[idx_d], add=True)

    plsc.subcore_barrier()
    readout(dg)


@functools.lru_cache(maxsize=None)
def _sc_edges():
  cp = pltpu.CompilerParams()
  if "needs_layout_passes" in pltpu.CompilerParams.__dataclass_fields__:
    cp = dataclasses.replace(cp, needs_layout_passes=False)
  return pl.kernel(
    _sc_body,
    out_type=(
        jax.ShapeDtypeStruct((NSC, N, D), _f32),
        jax.ShapeDtypeStruct((NSC, N, D), _f32),
        jax.ShapeDtypeStruct((NSC, N, D), _f32),
    ),
    mesh=plsc.VectorSubcoreMesh(core_axis_name="c", subcore_axis_name="s",
                                num_cores=NSC, num_subcores=NTILE),
    scratch_types=[
        pltpu.VMEM((C,), jnp.int32),
        pltpu.VMEM((C,), jnp.int32),
        pltpu.VMEM((C, D), _f32),
        pltpu.VMEM((C, D), _f32),
        pltpu.VMEM_SHARED((N, D), _f32),
    ],
    compiler_params=cp,
  )


def _post_body(x_ref, pa_ref, pv_ref, dg_ref, b_ref,
               was_ref, wvs_ref, ha_ref, hv_ref, g_ref,
               war_ref, wvr_ref, wp_ref, wo_ref, wval_ref,
               al_ref, nl_ref, val_ref, acc_a, acc_v, cnt_s):
    i = pl.program_id(0)

    @pl.when(i == 0)
    def _():
        acc_a[...] = jnp.zeros_like(acc_a)
        acc_v[...] = jnp.zeros_like(acc_v)
        cnt_s[...] = jnp.zeros_like(cnt_s)

    x = x_ref[...]
    recip = 1.0 / jnp.maximum(dg_ref[0, :, 0:1] + dg_ref[1, :, 0:1], 1.0)
    bidx = b_ref[...]
    oh = (bidx == lax.broadcasted_iota(jnp.int32, (1, B), 1)).astype(_f32)
    onesm = jnp.ones((NB, D), _f32)
    dn = (((0,), (0,)), ((), ()))

    agg_a = (pa_ref[0] + pa_ref[1]) * recip
    x2a = jnp.maximum(
        jnp.dot(x, was_ref[...], preferred_element_type=_f32) + agg_a
        + jnp.dot(oh, ha_ref[...], preferred_element_type=_f32), 0.0)
    acc_a[...] += lax.dot_general(oh, x2a, dn, preferred_element_type=_f32)
    nl_ref[...] = jnp.dot(x2a, wo_ref[...], preferred_element_type=_f32)

    agg_v = (pv_ref[0] + pv_ref[1]) * recip
    x2v = jnp.maximum(
        jnp.dot(x, wvs_ref[...], preferred_element_type=_f32) + agg_v
        + jnp.dot(oh, hv_ref[...], preferred_element_type=_f32), 0.0)
    acc_v[...] += lax.dot_general(oh, x2v, dn, preferred_element_type=_f32)

    cnt_s[...] += lax.dot_general(oh, onesm, dn, preferred_element_type=_f32)

    @pl.when(i == NBLK - 1)
    def _():
        cnt = jnp.maximum(cnt_s[...], 1.0)
        g = g_ref[...]
        ga = jnp.maximum(
            jnp.dot(acc_a[...] / cnt, war_ref[...], preferred_element_type=_f32) + g, 0.0)
        al_ref[...] = jnp.dot(ga, wp_ref[...], preferred_element_type=_f32)
        gv = jnp.maximum(
            jnp.dot(acc_v[...] / cnt, wvr_ref[...], preferred_element_type=_f32) + g, 0.0)
        val_ref[...] = jnp.dot(gv, wval_ref[...], preferred_element_type=_f32)


def _post(x, pa, pv, dg, bidx2, was, wvs, ha, hv, g, war, wvr, wp, wo, wval):
    full = lambda s: pl.BlockSpec(s, lambda i: tuple(0 for _ in s))
    return pl.pallas_call(
        _post_body,
        grid=(NBLK,),
        in_specs=[
            pl.BlockSpec((NB, D), lambda i: (i, 0)),
            pl.BlockSpec((NSC, NB, D), lambda i: (0, i, 0)),
            pl.BlockSpec((NSC, NB, D), lambda i: (0, i, 0)),
            pl.BlockSpec((NSC, NB, D), lambda i: (0, i, 0)),
            pl.BlockSpec((NB, 1), lambda i: (i, 0)),
            full((D, D)), full((D, D)), full((B, D)), full((B, D)), full((B, D)),
            full((D, D)), full((D, D)), full((D, A)), full((D, 1)), full((D, 1)),
        ],
        out_specs=[
            full((B, A)),
            pl.BlockSpec((NB, 1), lambda i: (i, 0)),
            full((B, 1)),
        ],
        out_shape=[
            jax.ShapeDtypeStruct((B, A), _f32),
            jax.ShapeDtypeStruct((N, 1), _f32),
            jax.ShapeDtypeStruct((B, 1), _f32),
        ],
        scratch_shapes=[
            pltpu.VMEM((B, D), _f32),
            pltpu.VMEM((B, D), _f32),
            pltpu.VMEM((B, D), _f32),
        ],
    )(x, pa, pv, dg, bidx2, was, wvs, ha, hv, g, war, wvr, wp, wo, wval)


def kernel(latent_nodes, latent_global, edge_attr, edge_index, batch_idx,
           Wa_msg, Wa_edge, Wa_self, Wa_glob, Wa_g, W_pred, W_obj,
           Wv_msg, Wv_edge, Wv_self, Wv_glob, Wv_g, W_val):
    x = latent_nodes.astype(_f32)
    g = latent_global.astype(_f32)
    ea = edge_attr.astype(_f32)
    ei = edge_index.astype(jnp.int32)
    bidx2 = batch_idx.astype(jnp.int32).reshape(N, 1)

    ya, yv, ha, hv = _prep(x, Wa_msg, Wv_msg, g, Wa_glob, Wv_glob)
    ta, tv = _eproj(ea, Wa_edge, Wv_edge)
    pa, pv, dg = _sc_edges()(ei[0], ei[1], ya, yv, ta, tv)
    al, nl, val = _post(x, pa, pv, dg, bidx2, Wa_self, Wv_self, ha, hv, g,
                        Wa_g, Wv_g, W_pred, W_obj, W_val)
    return (al, nl[:, 0], val[:, 0])


# speedup vs baseline: 3.7607x; 1.7319x over previous
import dataclasses
import functools

import jax
import jax.numpy as jnp
from jax import lax
from jax.experimental import pallas as pl
from jax.experimental.pallas import tpu as pltpu
from jax.experimental.pallas import tpu_sc as plsc

N = 10000
E = 320000
D = 128
DE = 16
B = 16
A = 32

NSC = 2
NTILE = 16
EPC = E // NSC
EPT = EPC // NTILE
C = 40
NCH = EPT // C
IB = 50
RS = 80
NSTR = N // RS
KMAX = (NSTR + NTILE - 1) // NTILE

NB = 1000
NBLK = N // NB
EB = 4000
EBLK = E // EB

_f32 = jnp.float32


def _prep_body(x_ref, wam_ref, wvm_ref, g_ref, wag_ref, wvg_ref,
               ya_ref, yv_ref, ha_ref, hv_ref):
    x = x_ref[...]
    ya_ref[...] = jnp.dot(x, wam_ref[...], preferred_element_type=_f32)
    yv_ref[...] = jnp.dot(x, wvm_ref[...], preferred_element_type=_f32)

    @pl.when(pl.program_id(0) == 0)
    def _():
        g = g_ref[...]
        ha_ref[...] = jnp.dot(g, wag_ref[...], preferred_element_type=_f32)
        hv_ref[...] = jnp.dot(g, wvg_ref[...], preferred_element_type=_f32)


def _prep(x, wam, wvm, g, wag, wvg):
    full = lambda s: pl.BlockSpec(s, lambda i: tuple(0 for _ in s))
    return pl.pallas_call(
        _prep_body,
        grid=(NBLK,),
        in_specs=[
            pl.BlockSpec((NB, D), lambda i: (i, 0)),
            full((D, D)), full((D, D)), full((B, D)), full((D, D)), full((D, D)),
        ],
        out_specs=[
            pl.BlockSpec((NB, D), lambda i: (i, 0)),
            pl.BlockSpec((NB, D), lambda i: (i, 0)),
            full((B, D)), full((B, D)),
        ],
        out_shape=[
            jax.ShapeDtypeStruct((N, D), _f32),
            jax.ShapeDtypeStruct((N, D), _f32),
            jax.ShapeDtypeStruct((B, D), _f32),
            jax.ShapeDtypeStruct((B, D), _f32),
        ],
    )(x, wam, wvm, g, wag, wvg)


def _eproj_body(ea_ref, wae_ref, wve_ref, ta_ref, tv_ref):
    ea = ea_ref[...]
    ta_ref[...] = jnp.dot(ea, wae_ref[...], preferred_element_type=_f32)
    tv_ref[...] = jnp.dot(ea, wve_ref[...], preferred_element_type=_f32)


def _eproj(ea, wae, wve):
    full = lambda s: pl.BlockSpec(s, lambda i: tuple(0 for _ in s))
    return pl.pallas_call(
        _eproj_body,
        grid=(EBLK,),
        in_specs=[
            pl.BlockSpec((EB, DE), lambda i: (i, 0)),
            full((DE, D)), full((DE, D)),
        ],
        out_specs=[
            pl.BlockSpec((EB, D), lambda i: (i, 0)),
            pl.BlockSpec((EB, D), lambda i: (i, 0)),
        ],
        out_shape=[
            jax.ShapeDtypeStruct((E, D), _f32),
            jax.ShapeDtypeStruct((E, D), _f32),
        ],
    )(ea, wae, wve)


def _sc_body(src2, dst2, ya, yv, ta, tv, pa, pv, dg,
             idx_sb, idx_db, rows0, rows1, trow0, trow1,
             sg0, sg1, st0, st1, ss0, ss1, acc_sh):
    cid = lax.axis_index("c")
    sid = lax.axis_index("s")
    zero16 = jnp.zeros((16,), _f32)
    e0 = (lax.iota(jnp.int32, 16) == 0).astype(_f32)
    bufs = ((rows0, trow0, sg0, st0, ss0), (rows1, trow1, sg1, st1, ss1))

    def zero_buf(r):
        @pl.loop(0, C)
        def _(i):
            for j in range(D // 16):
                r[i, pl.ds(j * 16, 16)] = zero16

    def for_stripes(fn):
        for k in range(KMAX):
            stripe = sid + NTILE * k

            @pl.when(stripe < NSTR)
            def _():
                fn(stripe * RS)

    def zero_acc(r0):
        pltpu.sync_copy(rows0, acc_sh.at[pl.ds(r0, C)])
        pltpu.sync_copy(rows0, acc_sh.at[pl.ds(r0 + C, C)])

    zero_buf(rows0)
    for_stripes(zero_acc)
    plsc.subcore_barrier()

    tchunk0 = (cid * EPC + sid * EPT) // C

    def edge_pass(y_hbm, t_hbm):
        def start_loads(b, crow, tbase):
            rows_b, trow_b, sg, st, _ = bufs[b]
            pltpu.make_async_copy(y_hbm.at[idx_sb.at[crow]], rows_b, sg).start()
            pltpu.make_async_copy(t_hbm.at[pl.ds(tbase, C)], trow_b, st).start()

        def wait_loads(b):
            rows_b, trow_b, sg, st, _ = bufs[b]
            pltpu.make_async_copy(y_hbm.at[idx_sb.at[0]], rows_b, sg).wait()
            pltpu.make_async_copy(t_hbm.at[pl.ds(0, C)], trow_b, st).wait()

        def wait_scatter(b):
            rows_b, _, _, _, ss = bufs[b]
            pltpu.make_async_copy(rows_b, acc_sh.at[idx_db.at[0]], ss).wait()

        for blk in range(NCH // IB):
            brow = tchunk0 + blk * IB
            pltpu.sync_copy(src2.at[cid, sid, blk], idx_sb)
            pltpu.sync_copy(dst2.at[cid, sid, blk], idx_db)
            start_loads(0, 0, brow * C)
            start_loads(1, 1, (brow + 1) * C)

            @pl.loop(0, IB // 2)
            def _(j2):
                for b in range(2):
                    cc = 2 * j2 + b
                    rows_b, trow_b, _, _, ss = bufs[b]
                    wait_loads(b)

                    @pl.loop(0, C)
                    def _(i):
                        for j in range(D // 16):
                            sl = pl.ds(j * 16, 16)
                            rows_b[i, sl] = jnp.maximum(
                                rows_b[i, sl] + trow_b[i, sl], 0.0)

                    pltpu.sync_copy(rows_b, acc_sh.at[idx_db.at[cc]], add=True)

                    @pl.when(j2 < IB // 2 - 1)
                    def _():
                        start_loads(b, cc + 2, (brow + cc + 2) * C)

    def readout(out_hbm):
        def cp(r0):
            sl = pl.ds(r0, RS)
            pltpu.sync_copy(acc_sh.at[sl], out_hbm.at[cid, sl])

        for_stripes(cp)

    edge_pass(ya, ta)
    plsc.subcore_barrier()
    readout(pa)
    zero_buf(rows0)
    for_stripes(zero_acc)
    plsc.subcore_barrier()

    edge_pass(yv, tv)
    plsc.subcore_barrier()
    readout(pv)
    zero_buf(rows0)
    for_stripes(zero_acc)
    plsc.subcore_barrier()

    @pl.loop(0, C)
    def _(i):
        trow0[i, pl.ds(0, 16)] = e0
        for j in range(1, D // 16):
            trow0[i, pl.ds(j * 16, 16)] = zero16

    for blk in range(NCH // IB):
        pltpu.sync_copy(dst2.at[cid, sid, blk], idx_db)

        @pl.loop(0, IB)
        def _(j1):
            pltpu.sync_copy(trow0, acc_sh.at[idx_db.at[j1]], add=True)

    plsc.subcore_barrier()
    readout(dg)


@functools.lru_cache(maxsize=None)
def _sc_edges():
  cp = pltpu.CompilerParams()
  if "needs_layout_passes" in pltpu.CompilerParams.__dataclass_fields__:
    cp = dataclasses.replace(cp, needs_layout_passes=False)
  return pl.kernel(
    _sc_body,
    out_type=(
        jax.ShapeDtypeStruct((NSC, N, D), _f32),
        jax.ShapeDtypeStruct((NSC, N, D), _f32),
        jax.ShapeDtypeStruct((NSC, N, D), _f32),
    ),
    mesh=plsc.VectorSubcoreMesh(core_axis_name="c", subcore_axis_name="s",
                                num_cores=NSC, num_subcores=NTILE),
    scratch_types=[
        pltpu.VMEM((IB, C), jnp.int32),
        pltpu.VMEM((IB, C), jnp.int32),
        pltpu.VMEM((C, D), _f32),
        pltpu.VMEM((C, D), _f32),
        pltpu.VMEM((C, D), _f32),
        pltpu.VMEM((C, D), _f32),
        pltpu.SemaphoreType.DMA,
        pltpu.SemaphoreType.DMA,
        pltpu.SemaphoreType.DMA,
        pltpu.SemaphoreType.DMA,
        pltpu.SemaphoreType.DMA,
        pltpu.SemaphoreType.DMA,
        pltpu.VMEM_SHARED((N, D), _f32),
    ],
    compiler_params=cp,
  )


def _post_body(x_ref, pa_ref, pv_ref, dg_ref, b_ref,
               was_ref, wvs_ref, ha_ref, hv_ref, g_ref,
               war_ref, wvr_ref, wp_ref, wo_ref, wval_ref,
               al_ref, nl_ref, val_ref, acc_a, acc_v, cnt_s):
    i = pl.program_id(0)

    @pl.when(i == 0)
    def _():
        acc_a[...] = jnp.zeros_like(acc_a)
        acc_v[...] = jnp.zeros_like(acc_v)
        cnt_s[...] = jnp.zeros_like(cnt_s)

    x = x_ref[...]
    recip = 1.0 / jnp.maximum(dg_ref[0, :, 0:1] + dg_ref[1, :, 0:1], 1.0)
    bidx = b_ref[...]
    oh = (bidx == lax.broadcasted_iota(jnp.int32, (1, B), 1)).astype(_f32)
    onesm = jnp.ones((NB, D), _f32)
    dn = (((0,), (0,)), ((), ()))

    agg_a = (pa_ref[0] + pa_ref[1]) * recip
    x2a = jnp.maximum(
        jnp.dot(x, was_ref[...], preferred_element_type=_f32) + agg_a
        + jnp.dot(oh, ha_ref[...], preferred_element_type=_f32), 0.0)
    acc_a[...] += lax.dot_general(oh, x2a, dn, preferred_element_type=_f32)
    nl_ref[...] = jnp.dot(x2a, wo_ref[...], preferred_element_type=_f32)

    agg_v = (pv_ref[0] + pv_ref[1]) * recip
    x2v = jnp.maximum(
        jnp.dot(x, wvs_ref[...], preferred_element_type=_f32) + agg_v
        + jnp.dot(oh, hv_ref[...], preferred_element_type=_f32), 0.0)
    acc_v[...] += lax.dot_general(oh, x2v, dn, preferred_element_type=_f32)

    cnt_s[...] += lax.dot_general(oh, onesm, dn, preferred_element_type=_f32)

    @pl.when(i == NBLK - 1)
    def _():
        cnt = jnp.maximum(cnt_s[...], 1.0)
        g = g_ref[...]
        ga = jnp.maximum(
            jnp.dot(acc_a[...] / cnt, war_ref[...], preferred_element_type=_f32) + g, 0.0)
        al_ref[...] = jnp.dot(ga, wp_ref[...], preferred_element_type=_f32)
        gv = jnp.maximum(
            jnp.dot(acc_v[...] / cnt, wvr_ref[...], preferred_element_type=_f32) + g, 0.0)
        val_ref[...] = jnp.dot(gv, wval_ref[...], preferred_element_type=_f32)


def _post(x, pa, pv, dg, bidx2, was, wvs, ha, hv, g, war, wvr, wp, wo, wval):
    full = lambda s: pl.BlockSpec(s, lambda i: tuple(0 for _ in s))
    return pl.pallas_call(
        _post_body,
        grid=(NBLK,),
        in_specs=[
            pl.BlockSpec((NB, D), lambda i: (i, 0)),
            pl.BlockSpec((NSC, NB, D), lambda i: (0, i, 0)),
            pl.BlockSpec((NSC, NB, D), lambda i: (0, i, 0)),
            pl.BlockSpec((NSC, NB, D), lambda i: (0, i, 0)),
            pl.BlockSpec((NB, 1), lambda i: (i, 0)),
            full((D, D)), full((D, D)), full((B, D)), full((B, D)), full((B, D)),
            full((D, D)), full((D, D)), full((D, A)), full((D, 1)), full((D, 1)),
        ],
        out_specs=[
            full((B, A)),
            pl.BlockSpec((NB, 1), lambda i: (i, 0)),
            full((B, 1)),
        ],
        out_shape=[
            jax.ShapeDtypeStruct((B, A), _f32),
            jax.ShapeDtypeStruct((N, 1), _f32),
            jax.ShapeDtypeStruct((B, 1), _f32),
        ],
        scratch_shapes=[
            pltpu.VMEM((B, D), _f32),
            pltpu.VMEM((B, D), _f32),
            pltpu.VMEM((B, D), _f32),
        ],
    )(x, pa, pv, dg, bidx2, was, wvs, ha, hv, g, war, wvr, wp, wo, wval)


def kernel(latent_nodes, latent_global, edge_attr, edge_index, batch_idx,
           Wa_msg, Wa_edge, Wa_self, Wa_glob, Wa_g, W_pred, W_obj,
           Wv_msg, Wv_edge, Wv_self, Wv_glob, Wv_g, W_val):
    x = latent_nodes.astype(_f32)
    g = latent_global.astype(_f32)
    ea = edge_attr.astype(_f32)
    ei = edge_index.astype(jnp.int32)
    bidx2 = batch_idx.astype(jnp.int32).reshape(N, 1)

    ya, yv, ha, hv = _prep(x, Wa_msg, Wv_msg, g, Wa_glob, Wv_glob)
    ta, tv = _eproj(ea, Wa_edge, Wv_edge)
    src2 = ei[0].reshape(NSC, NTILE, NCH // IB, IB, C)
    dst2 = ei[1].reshape(NSC, NTILE, NCH // IB, IB, C)
    pa, pv, dg = _sc_edges()(src2, dst2, ya, yv, ta, tv)
    al, nl, val = _post(x, pa, pv, dg, bidx2, Wa_self, Wv_self, ha, hv, g,
                        Wa_g, Wv_g, W_pred, W_obj, W_val)
    return (al, nl[:, 0], val[:, 0])
